# baseline (speedup 1.0000x reference)
import functools

import jax
import jax.numpy as jnp
from jax import lax
from jax.experimental import pallas as pl
from jax.experimental.pallas import tpu as pltpu
from jax.experimental.pallas import tpu_sc as plsc

N_GENES = 100000
N_EMBED = 64
BATCH = 4096
SEQ = 200
NPARTS = 2
SEQP = SEQ // NPARTS

NW = 32
BPW = BATCH // NW
NCH = BPW
NBUF = 3
LANES = 16
VPR = N_EMBED // LANES
NGRP = SEQP // LANES
TAIL0 = SEQP - LANES
TAILI = LANES - (SEQP - NGRP * LANES)
PROWS = SEQP // 2

_C = (
    0.04342890782205806,
    -0.40486717441854486,
    1.5939013634971635,
    -3.4924942798763934,
    5.0468760449737635,
    -2.7868129538668147,
)
_LN2 = 0.6931471805599453


def _log1p16(x):
    t = x + jnp.float32(1.0)
    ti = lax.bitcast_convert_type(t, jnp.int32)
    e = (ti >> 23) - 127
    mi = (ti & jnp.int32(0x007FFFFF)) | jnp.int32(0x3F800000)
    m = lax.bitcast_convert_type(mi, jnp.float32)
    p = jnp.full((LANES,), _C[0], jnp.float32)
    for c in _C[1:]:
        p = p * m + jnp.float32(c)
    return (e.astype(jnp.float32) + p) * jnp.float32(_LN2)


_DNUMS = lax.GatherDimensionNumbers(
    offset_dims=(), collapsed_slice_dims=(0,), start_index_map=(0,)
)


def _bcast(vec, i):
    return lax.gather(
        vec,
        jnp.full((LANES, 1), i, jnp.int32),
        _DNUMS,
        slice_sizes=(1,),
        mode=lax.GatherScatterMode.PROMISE_IN_BOUNDS,
    )


def _sc_body(table, genes, counts, out, idx_v, cnt_v, gbufs, pbufs, gsem, osem):
    cid = lax.axis_index("c")
    sid = lax.axis_index("s")
    wid = sid * 2 + cid
    b0 = wid * BPW

    pltpu.sync_copy(genes.at[pl.ds(b0, BPW)], idx_v)
    pltpu.sync_copy(counts.at[pl.ds(b0, BPW)], cnt_v)

    def start_gather(k, b):
        pltpu.async_copy(table.at[idx_v.at[k]], gbufs[b], gsem[b])

    def wait_gather(b):
        pltpu.make_async_copy(table.at[idx_v.at[0]], gbufs[b], gsem[b]).wait()

    def start_out(k, b):
        pltpu.async_copy(pbufs[b], out.at[pl.ds((b0 + k) * PROWS, PROWS)], osem[b])

    def wait_out(b):
        pltpu.make_async_copy(pbufs[b], out.at[pl.ds(0, PROWS)], osem[b]).wait()

    def scale(k, b):
        gb = gbufs[b]
        pb = pbufs[b]

        @plsc.parallel_loop(0, NGRP)
        def _(g):
            lp_vec = _log1p16(cnt_v[k, pl.ds(g * LANES, LANES)])
            for i in range(LANES):
                s = g * LANES + i
                prow = g * (LANES // 2) + i // 2
                pcol = (i % 2) * N_EMBED
                sc = _bcast(lp_vec, i)
                for j in range(VPR):
                    pb[prow, pl.ds(pcol + j * LANES, LANES)] = (
                        gb[s, pl.ds(j * LANES, LANES)] * sc
                    )

        lp_vec = _log1p16(cnt_v[k, pl.ds(TAIL0, LANES)])
        for i in range(TAILI, LANES):
            s = TAIL0 + i
            prow = s // 2
            pcol = (s % 2) * N_EMBED
            sc = _bcast(lp_vec, i)
            for j in range(VPR):
                pb[prow, pl.ds(pcol + j * LANES, LANES)] = (
                    gb[s, pl.ds(j * LANES, LANES)] * sc
                )

    def step(k, b):
        wait_gather(b)

        @pl.when(k >= NBUF)
        def _():
            wait_out(b)

        scale(k, b)
        start_out(k, b)

        @pl.when(k + NBUF < NCH)
        def _():
            start_gather(k + NBUF, b)

    for k in range(NBUF):
        start_gather(k, k)

    def tri_body(i, carry):
        for j in range(NBUF):
            step(i * NBUF + j, j)
        return carry

    lax.fori_loop(0, NCH // NBUF, tri_body, 0)
    for k in range(NCH - NCH % NBUF, NCH):
        step(k, k % NBUF)
    for k in range(NCH - NBUF, NCH):
        wait_out(k % NBUF)


def _make_sc():
    mesh = plsc.VectorSubcoreMesh(core_axis_name="c", subcore_axis_name="s")

    def body(table, genes_, counts_, out, idx_v, cnt_v, gb0, gb1, gb2,
             pb0, pb1, pb2, g0, g1, g2, o0, o1, o2):
        _sc_body(
            table, genes_, counts_, out, idx_v, cnt_v,
            (gb0, gb1, gb2), (pb0, pb1, pb2),
            (g0, g1, g2), (o0, o1, o2),
        )

    return pl.kernel(
        body,
        mesh=mesh,
        compiler_params=pltpu.CompilerParams(use_tc_tiling_on_sc=False),
        out_type=jax.ShapeDtypeStruct((BATCH * SEQP // 2, 2 * N_EMBED), jnp.float32),
        scratch_types=[
            pltpu.VMEM((BPW, SEQP), jnp.int32),
            pltpu.VMEM((BPW, SEQP), jnp.float32),
        ]
        + [pltpu.VMEM((SEQP, N_EMBED), jnp.float32)] * NBUF
        + [pltpu.VMEM((PROWS, 2 * N_EMBED), jnp.float32)] * NBUF
        + [pltpu.SemaphoreType.DMA] * (2 * NBUF),
    )


def _run(counts, genes, gene_embedding):
    sc = _make_sc()
    parts = []
    for p in range(NPARTS):
        gslice = lax.slice_in_dim(genes, p * SEQP, (p + 1) * SEQP, axis=1)
        cslice = lax.slice_in_dim(counts, p * SEQP, (p + 1) * SEQP, axis=1)
        packed = sc(gene_embedding, gslice, cslice)
        parts.append(packed.reshape(BATCH, SEQP, N_EMBED))
    return jnp.concatenate(parts, axis=1)


def kernel(counts, genes, gene_embedding):
    return _run(counts, genes.astype(jnp.int32), gene_embedding)


# speedup vs baseline: 1.3573x; 1.3573x over previous
import jax
import jax.numpy as jnp
from jax import lax
from jax.experimental import pallas as pl
from jax.experimental.pallas import tpu as pltpu
from jax.experimental.pallas import tpu_sc as plsc
from jax.experimental.layout import with_layout_constraint, Format, Layout

N_GENES = 100000
N_EMBED = 64
BATCH = 4096
SEQ = 200

NW = 32
BPW = BATCH // NW
NCH = BPW
NBUF = 3
LANES = 16
VPR = N_EMBED // LANES
NGRP = SEQ // LANES
TAIL0 = SEQ - LANES
TAILI = LANES - (SEQ - NGRP * LANES)
PROWS = SEQ // 2

_C = (
    0.04342890782205806,
    -0.40486717441854486,
    1.5939013634971635,
    -3.4924942798763934,
    5.0468760449737635,
    -2.7868129538668147,
)
_LN2 = 0.6931471805599453


def _log1p16(x):
    t = x + jnp.float32(1.0)
    ti = lax.bitcast_convert_type(t, jnp.int32)
    e = (ti >> 23) - 127
    mi = (ti & jnp.int32(0x007FFFFF)) | jnp.int32(0x3F800000)
    m = lax.bitcast_convert_type(mi, jnp.float32)
    p = jnp.full((LANES,), _C[0], jnp.float32)
    for c in _C[1:]:
        p = p * m + jnp.float32(c)
    return (e.astype(jnp.float32) + p) * jnp.float32(_LN2)


_DNUMS = lax.GatherDimensionNumbers(
    offset_dims=(), collapsed_slice_dims=(0,), start_index_map=(0,)
)


def _bcast(vec, i):
    return lax.gather(
        vec,
        jnp.full((LANES, 1), i, jnp.int32),
        _DNUMS,
        slice_sizes=(1,),
        mode=lax.GatherScatterMode.PROMISE_IN_BOUNDS,
    )


def _sc_body(table, genes, counts, out, idx_v, cnt_v, gbufs, pbufs, gsem, osem):
    cid = lax.axis_index("c")
    sid = lax.axis_index("s")
    wid = sid * 2 + cid
    b0 = wid * BPW

    pltpu.sync_copy(genes.at[pl.ds(b0, BPW)], idx_v)
    pltpu.sync_copy(counts.at[pl.ds(b0, BPW)], cnt_v)

    def start_gather(k, b):
        pltpu.async_copy(table.at[idx_v.at[k]], gbufs[b], gsem[b])

    def wait_gather(b):
        pltpu.make_async_copy(table.at[idx_v.at[0]], gbufs[b], gsem[b]).wait()

    def start_out(k, b):
        pltpu.async_copy(pbufs[b], out.at[pl.ds((b0 + k) * PROWS, PROWS)], osem[b])

    def wait_out(b):
        pltpu.make_async_copy(pbufs[b], out.at[pl.ds(0, PROWS)], osem[b]).wait()

    def scale(k, b):
        gb = gbufs[b]
        pb = pbufs[b]

        @plsc.parallel_loop(0, NGRP)
        def _(g):
            lp_vec = _log1p16(cnt_v[k, pl.ds(g * LANES, LANES)])
            for i in range(LANES):
                s = g * LANES + i
                prow = g * (LANES // 2) + i // 2
                pcol = (i % 2) * N_EMBED
                sc = _bcast(lp_vec, i)
                for j in range(VPR):
                    pb[prow, pl.ds(pcol + j * LANES, LANES)] = (
                        gb[s, pl.ds(j * LANES, LANES)] * sc
                    )

        lp_vec = _log1p16(cnt_v[k, pl.ds(TAIL0, LANES)])
        for i in range(TAILI, LANES):
            s = TAIL0 + i
            prow = s // 2
            pcol = (s % 2) * N_EMBED
            sc = _bcast(lp_vec, i)
            for j in range(VPR):
                pb[prow, pl.ds(pcol + j * LANES, LANES)] = (
                    gb[s, pl.ds(j * LANES, LANES)] * sc
                )

    def step(k, b):
        wait_gather(b)

        @pl.when(k >= NBUF)
        def _():
            wait_out(b)

        scale(k, b)
        start_out(k, b)

        @pl.when(k + NBUF < NCH)
        def _():
            start_gather(k + NBUF, b)

    for k in range(NBUF):
        start_gather(k, k)

    def tri_body(i, carry):
        for j in range(NBUF):
            step(i * NBUF + j, j)
        return carry

    lax.fori_loop(0, NCH // NBUF, tri_body, 0)
    for k in range(NCH - NCH % NBUF, NCH):
        step(k, k % NBUF)
    for k in range(NCH - NBUF, NCH):
        wait_out(k % NBUF)


def _run(counts, genes, gene_embedding):
    mesh = plsc.VectorSubcoreMesh(core_axis_name="c", subcore_axis_name="s")

    def body(table, genes_, counts_, out, idx_v, cnt_v, gb0, gb1, gb2,
             pb0, pb1, pb2, g0, g1, g2, o0, o1, o2):
        _sc_body(
            table, genes_, counts_, out, idx_v, cnt_v,
            (gb0, gb1, gb2), (pb0, pb1, pb2),
            (g0, g1, g2), (o0, o1, o2),
        )

    sc = pl.kernel(
        body,
        mesh=mesh,
        compiler_params=pltpu.CompilerParams(use_tc_tiling_on_sc=False),
        out_type=jax.ShapeDtypeStruct((BATCH * SEQ // 2, 2 * N_EMBED), jnp.float32),
        scratch_types=[
            pltpu.VMEM((BPW, SEQ), jnp.int32),
            pltpu.VMEM((BPW, SEQ), jnp.float32),
        ]
        + [pltpu.VMEM((SEQ, N_EMBED), jnp.float32)] * NBUF
        + [pltpu.VMEM((PROWS, 2 * N_EMBED), jnp.float32)] * NBUF
        + [pltpu.SemaphoreType.DMA] * (2 * NBUF),
    )
    packed = sc(gene_embedding, genes, counts)
    out3 = packed.reshape(BATCH, SEQ, N_EMBED)
    out3 = with_layout_constraint(out3, Layout((1, 2, 0), tiling=((8, 128),)))
    return out3


def kernel(counts, genes, gene_embedding):
    return _run(counts, genes.astype(jnp.int32), gene_embedding)
